# Initial kernel scaffold; baseline (speedup 1.0000x reference)
import jax
import jax.numpy as jnp
from jax.experimental import pallas as pl


def kernel(x, edge_index, emb, node_len, params):
    raise NotImplementedError("write your pallas kernel here")


# speedup vs baseline: 1.1238x; 1.1238x over previous
import functools
import jax
import jax.numpy as jnp
from jax import lax
from jax.experimental import pallas as pl

_N = 10000
_HS = 128


def _embed_kernel(idx_ref, tr_ref, tg_ref, or_ref, og_ref):
    idx = idx_ref[...]
    nt = tr_ref.shape[0]
    oh = (idx == lax.broadcasted_iota(jnp.int32, (idx.shape[0], nt), 1)).astype(jnp.float32)
    or_ref[...] = jnp.dot(oh, tr_ref[...], preferred_element_type=jnp.float32)
    og_ref[...] = jnp.dot(oh, tg_ref[...], preferred_element_type=jnp.float32)


def _embed_lookup(x, tr, tg):
    n = x.shape[0]
    bm = 2000
    grid = n // bm
    return pl.pallas_call(
        _embed_kernel,
        grid=(grid,),
        in_specs=[
            pl.BlockSpec((bm, 1), lambda i: (i, 0)),
            pl.BlockSpec(tr.shape, lambda i: (0, 0)),
            pl.BlockSpec(tg.shape, lambda i: (0, 0)),
        ],
        out_specs=[
            pl.BlockSpec((bm, _HS), lambda i: (i, 0)),
            pl.BlockSpec((bm, _HS), lambda i: (i, 0)),
        ],
        out_shape=[
            jax.ShapeDtypeStruct((n, _HS), jnp.float32),
            jax.ShapeDtypeStruct((n, _HS), jnp.float32),
        ],
    )(x, tr, tg)


def _proj_kernel(h_ref, wt_ref, a2_ref, hh_ref, s_ref):
    hh = jnp.dot(h_ref[...], wt_ref[...], preferred_element_type=jnp.float32)
    hh_ref[...] = hh
    s_ref[...] = jnp.dot(hh, a2_ref[...], preferred_element_type=jnp.float32)


def _gat_project(h, wt, a2):
    n, k = h.shape
    ho = wt.shape[1]
    bm = 1000
    grid = n // bm
    return pl.pallas_call(
        _proj_kernel,
        grid=(grid,),
        in_specs=[
            pl.BlockSpec((bm, k), lambda i: (i, 0)),
            pl.BlockSpec((k, ho), lambda i: (0, 0)),
            pl.BlockSpec((ho, 128), lambda i: (0, 0)),
        ],
        out_specs=[
            pl.BlockSpec((bm, ho), lambda i: (i, 0)),
            pl.BlockSpec((bm, 128), lambda i: (i, 0)),
        ],
        out_shape=[
            jax.ShapeDtypeStruct((n, ho), jnp.float32),
            jax.ShapeDtypeStruct((n, 128), jnp.float32),
        ],
    )(h, wt, a2)


def _le_kernel(e_ref, w_ref, b_ref, o_ref):
    y = jnp.dot(e_ref[...], w_ref[...], preferred_element_type=jnp.float32)
    o_ref[...] = jnp.maximum(y + b_ref[0, :][None, :], 0.0)


def _le_project(emb, wt, b_pad):
    n, k = emb.shape
    bm = 2000
    return pl.pallas_call(
        _le_kernel,
        grid=(n // bm,),
        in_specs=[
            pl.BlockSpec((bm, k), lambda i: (i, 0)),
            pl.BlockSpec((k, _HS), lambda i: (0, 0)),
            pl.BlockSpec((8, _HS), lambda i: (0, 0)),
        ],
        out_specs=pl.BlockSpec((bm, _HS), lambda i: (i, 0)),
        out_shape=jax.ShapeDtypeStruct((n, _HS), jnp.float32),
    )(emb, wt, b_pad)


def _mean_kernel(h_ref, o_ref):
    @pl.when(pl.program_id(0) == 0)
    def _():
        o_ref[...] = jnp.zeros_like(o_ref)

    o_ref[...] += jnp.sum(h_ref[...], axis=0, keepdims=True) / _N


def _node_mean(h):
    n, c = h.shape
    bm = 2000
    out = pl.pallas_call(
        _mean_kernel,
        grid=(n // bm,),
        in_specs=[pl.BlockSpec((bm, c), lambda i: (i, 0))],
        out_specs=pl.BlockSpec((1, c), lambda i: (0, 0)),
        out_shape=jax.ShapeDtypeStruct((1, c), jnp.float32),
    )(h)
    return out[0]


def _cnn_kernel(h_ref, xr_ref, ep_ref, m_ref, wtaps_ref, bsum_ref,
                l1t_ref, l1b_ref, l2t_ref, l2b_ref,
                ocnn_ref, ograph_ref, eseq_ref, *, seg, taps):
    pad = 512
    maxp = (max(taps) - 1) // 2
    mask = m_ref[0, 0, :][:, None]
    xin = (xr_ref[0] + ep_ref[0]) * 0.5
    xin = jnp.concatenate([xin, jnp.zeros((pad - seg, _HS), jnp.float32)], axis=0)
    xin = xin * mask
    xp = jnp.concatenate([
        jnp.zeros((maxp, _HS), jnp.float32), xin,
        jnp.zeros((maxp, _HS), jnp.float32)], axis=0)
    y = jnp.zeros((pad, 64), jnp.float32)
    j = 0
    for t in taps:
        p = (t - 1) // 2
        for k in range(t):
            y = y + jnp.dot(xp[maxp - p + k:maxp - p + k + pad, :],
                            wtaps_ref[j], preferred_element_type=jnp.float32)
            j += 1
    xm = (y + bsum_ref[0, :64][None, :]) / 3.0
    o1 = jnp.maximum(
        jnp.dot(xm, l1t_ref[...], preferred_element_type=jnp.float32)
        + l1b_ref[0, :][None, :], 0.0)
    ocnn = (jnp.dot(o1, l2t_ref[...], preferred_element_type=jnp.float32)
            + l2b_ref[0, :][None, :])
    ocnn_ref[0] = ocnn
    hg = jnp.concatenate([h_ref[0], jnp.zeros((pad - seg, _HS), jnp.float32)], axis=0)
    ograph_ref[0] = hg * mask
    eseq_ref[0] = jnp.sum(ocnn * mask, axis=0, keepdims=True) / pad


def _cnn_stage(h3, x_r, embp, mask3, wtaps, bsum, l1t, l1b, l2t, l2b, nb, seg, taps):
    kfn = functools.partial(_cnn_kernel, seg=seg, taps=taps)
    ntap = wtaps.shape[0]
    return pl.pallas_call(
        kfn,
        grid=(nb,),
        in_specs=[
            pl.BlockSpec((1, seg, _HS), lambda b: (b, 0, 0)),
            pl.BlockSpec((1, seg, _HS), lambda b: (b, 0, 0)),
            pl.BlockSpec((1, seg, _HS), lambda b: (b, 0, 0)),
            pl.BlockSpec((1, 1, 512), lambda b: (b, 0, 0)),
            pl.BlockSpec((ntap, _HS, 64), lambda b: (0, 0, 0)),
            pl.BlockSpec((8, 128), lambda b: (0, 0)),
            pl.BlockSpec((64, 512), lambda b: (0, 0)),
            pl.BlockSpec((8, 512), lambda b: (0, 0)),
            pl.BlockSpec((512, 128), lambda b: (0, 0)),
            pl.BlockSpec((8, 128), lambda b: (0, 0)),
        ],
        out_specs=[
            pl.BlockSpec((1, 512, _HS), lambda b: (b, 0, 0)),
            pl.BlockSpec((1, 512, _HS), lambda b: (b, 0, 0)),
            pl.BlockSpec((1, 1, _HS), lambda b: (b, 0, 0)),
        ],
        out_shape=[
            jax.ShapeDtypeStruct((nb, 512, _HS), jnp.float32),
            jax.ShapeDtypeStruct((nb, 512, _HS), jnp.float32),
            jax.ShapeDtypeStruct((nb, 1, _HS), jnp.float32),
        ],
    )(h3, x_r, embp, mask3, wtaps, bsum, l1t, l1b, l2t, l2b)


def _edge_stage(s, hh, src, dst, heads, outc, bound):
    al = s[src, 0:heads] + s[dst, heads:2 * heads]
    al = jnp.where(al > 0, al, 0.2 * al)
    e = jnp.exp(al - bound)
    den = jax.ops.segment_sum(e, dst, num_segments=_N)
    a = e / den[dst]
    hh3 = hh.reshape(_N, heads, outc)
    out = jax.ops.segment_sum(hh3[src] * a[:, :, None], dst, num_segments=_N)
    return out


def _pad8(v, w):
    return jnp.zeros((8, w), jnp.float32).at[0, :v.shape[0]].set(v)


def kernel(x, edge_index, emb, node_len, params):
    p = params
    nb = node_len.shape[0]
    seg = _N // nb
    loop = jnp.arange(_N, dtype=edge_index.dtype)
    src = jnp.concatenate([edge_index[0], loop])
    dst = jnp.concatenate([edge_index[1], loop])

    tr = jnp.zeros((8, _HS), jnp.float32).at[:6].set(p['x_emb_r'])
    tg = jnp.zeros((8, _HS), jnp.float32).at[:6].set(p['x_emb_g'])
    x_r, x_g = _embed_lookup(x, tr, tg)

    h = x_g
    for li, (wn, heads, outc) in enumerate(
            [('g1', 4, 256), ('g2', 4, 256), ('g3', 1, 128)]):
        W = p[wn + '_W']
        a_s = p[wn + '_as'][0]
        a_d = p[wn + '_ad'][0]
        b = p[wn + '_b']
        ho = W.shape[0]
        a2 = jnp.zeros((ho, 128), jnp.float32)
        for hd in range(heads):
            a2 = a2.at[hd * outc:(hd + 1) * outc, hd].set(a_s[hd])
            a2 = a2.at[hd * outc:(hd + 1) * outc, heads + hd].set(a_d[hd])
        hh, s = _gat_project(h, W.T, a2)
        bound = jnp.max(s[:, 0:heads]) + jnp.max(s[:, heads:2 * heads])
        bound = jnp.where(bound > 0, bound, 0.2 * bound)
        out = _edge_stage(s, hh, src, dst, heads, outc, bound)
        h = jnp.maximum(out.mean(axis=1) + b[None, :], 0.0)

    emb_graph = _node_mean(h)
    embp = _le_project(emb, p['le_W'].T, _pad8(p['le_b'], _HS))

    mask = (jnp.arange(512, dtype=node_len.dtype)[None, :]
            < node_len[:, None]).astype(jnp.float32)
    mask3 = mask.reshape(nb, 1, 512)

    taps = (7, 11, 15)
    wtaps = jnp.concatenate([
        jnp.transpose(p['c1_W'], (2, 1, 0)),
        jnp.transpose(p['c2_W'], (2, 1, 0)),
        jnp.transpose(p['c3_W'], (2, 1, 0))], axis=0)
    bsum = _pad8(p['c1_b'] + p['c2_b'] + p['c3_b'], 128)

    out_cnn, out_graph, emb_seq3 = _cnn_stage(
        h.reshape(nb, seg, _HS), x_r.reshape(nb, seg, _HS),
        embp.reshape(nb, seg, _HS), mask3,
        wtaps, bsum, p['l1_W'].T, _pad8(p['l1_b'], 512),
        p['l2_W'].T, _pad8(p['l2_b'], _HS), nb, seg, taps)

    return (out_cnn, out_graph, mask, mask, emb_seq3[:, 0, :], emb_graph)
